# baseline (speedup 1.0000x reference)
import functools

import jax
import jax.numpy as jnp
from jax import lax
from jax.experimental import pallas as pl
from jax.experimental.pallas import tpu as pltpu
from jax.experimental.pallas import tpu_sc as plsc

D = 32
NC, NS = 2, 16
NW = NC * NS
CHUNK = 1024


def _make_gather(b_total: int):
    b_per_w = b_total // NW
    n_chunk = b_per_w // CHUNK
    assert n_chunk % 2 == 0
    mesh = plsc.VectorSubcoreMesh(core_axis_name="c", subcore_axis_name="s")

    @functools.partial(
        pl.kernel,
        mesh=mesh,
        compiler_params=pltpu.CompilerParams(use_tc_tiling_on_sc=False),
        out_type=jax.ShapeDtypeStruct((b_total, D), jnp.float32),
        scratch_types=[
            pltpu.VMEM((CHUNK,), jnp.int32),
            pltpu.VMEM((CHUNK,), jnp.int32),
            pltpu.VMEM((CHUNK, D), jnp.float32),
            pltpu.VMEM((CHUNK, D), jnp.float32),
            pltpu.SemaphoreType.DMA,
            pltpu.SemaphoreType.DMA,
            pltpu.SemaphoreType.DMA,
            pltpu.SemaphoreType.DMA,
            pltpu.SemaphoreType.DMA,
            pltpu.SemaphoreType.DMA,
        ],
    )
    def gather_kernel(idx_hbm, table_hbm, out_hbm,
                      idx0, idx1, rows0, rows1,
                      si0, si1, sg0, sg1, ss0, ss1):
        wid = lax.axis_index("s") * NC + lax.axis_index("c")
        wbase = wid * b_per_w
        idx_v = (idx0, idx1)
        rows_v = (rows0, rows1)
        sem_i = (si0, si1)
        sem_g = (sg0, sg1)
        sem_s = (ss0, ss1)

        def idx_copy(i, b):
            return pltpu.make_async_copy(
                idx_hbm.at[pl.ds(wbase + i * CHUNK, CHUNK)], idx_v[b], sem_i[b])

        def gather_copy(b):
            return pltpu.make_async_copy(
                table_hbm.at[idx_v[b]], rows_v[b], sem_g[b])

        def store_copy(i, b):
            return pltpu.make_async_copy(
                rows_v[b], out_hbm.at[pl.ds(wbase + i * CHUNK, CHUNK)], sem_s[b])

        idx_copy(0, 0).start()
        idx_copy(1, 1).start()
        idx_copy(0, 0).wait()
        gather_copy(0).start()

        def pipe_step(i, b, nb):
            gather_copy(b).wait()
            store_copy(i, b).start()

            @pl.when(i + 1 < n_chunk)
            def _():
                idx_copy(i + 1, nb).wait()

                @pl.when(i >= 1)
                def _():
                    store_copy(i - 1, nb).wait()
                gather_copy(nb).start()

                @pl.when(i + 2 < n_chunk)
                def _():
                    idx_copy(i + 2, b).start()

        def body(j, _):
            pipe_step(2 * j, 0, 1)
            pipe_step(2 * j + 1, 1, 0)
            return 0

        lax.fori_loop(0, n_chunk // 2, body, 0)

        store_copy(n_chunk - 2, (n_chunk - 2) % 2).wait()
        store_copy(n_chunk - 1, (n_chunk - 1) % 2).wait()

    return gather_kernel


def kernel(chars, table):
    b, s = chars.shape
    idx = chars.reshape(-1).astype(jnp.int32)
    out = _make_gather(b * s)(idx, table)
    return out.reshape(b, s, D)


# speedup vs baseline: 1.3794x; 1.3794x over previous
import functools

import jax
import jax.numpy as jnp
from jax import lax
from jax.experimental import pallas as pl
from jax.experimental.pallas import tpu as pltpu
from jax.experimental.pallas import tpu_sc as plsc

D = 32
NC, NS = 2, 16
NW = NC * NS
CHUNK = 1024


def _make_gather(b_total: int):
    b_per_w = b_total // NW
    n_chunk = b_per_w // CHUNK
    assert n_chunk % 2 == 0
    mesh = plsc.VectorSubcoreMesh(core_axis_name="c", subcore_axis_name="s")

    @functools.partial(
        pl.kernel,
        mesh=mesh,
        compiler_params=pltpu.CompilerParams(use_tc_tiling_on_sc=False),
        out_type=jax.ShapeDtypeStruct((b_total, D), jnp.float32),
        scratch_types=[
            pltpu.VMEM_SHARED((1000, D), jnp.float32),
            pltpu.VMEM((CHUNK,), jnp.int32),
            pltpu.VMEM((CHUNK,), jnp.int32),
            pltpu.VMEM((CHUNK, D), jnp.float32),
            pltpu.VMEM((CHUNK, D), jnp.float32),
            pltpu.SemaphoreType.DMA,
            pltpu.SemaphoreType.DMA,
            pltpu.SemaphoreType.DMA,
            pltpu.SemaphoreType.DMA,
            pltpu.SemaphoreType.DMA,
            pltpu.SemaphoreType.DMA,
        ],
    )
    def gather_kernel(idx_hbm, table_hbm, out_hbm,
                      table_sh, idx0, idx1, rows0, rows1,
                      si0, si1, sg0, sg1, ss0, ss1):
        wid = lax.axis_index("s") * NC + lax.axis_index("c")
        wbase = wid * b_per_w

        @pl.when(lax.axis_index("s") == 0)
        def _():
            pltpu.sync_copy(table_hbm, table_sh)
        plsc.subcore_barrier()
        idx_v = (idx0, idx1)
        rows_v = (rows0, rows1)
        sem_i = (si0, si1)
        sem_g = (sg0, sg1)
        sem_s = (ss0, ss1)

        def idx_copy(i, b):
            return pltpu.make_async_copy(
                idx_hbm.at[pl.ds(wbase + i * CHUNK, CHUNK)], idx_v[b], sem_i[b])

        def gather_copy(b):
            return pltpu.make_async_copy(
                table_sh.at[idx_v[b]], rows_v[b], sem_g[b])

        def store_copy(i, b):
            return pltpu.make_async_copy(
                rows_v[b], out_hbm.at[pl.ds(wbase + i * CHUNK, CHUNK)], sem_s[b])

        idx_copy(0, 0).start()
        idx_copy(1, 1).start()
        idx_copy(0, 0).wait()
        gather_copy(0).start()

        def pipe_step(i, b, nb):
            gather_copy(b).wait()
            store_copy(i, b).start()

            @pl.when(i + 1 < n_chunk)
            def _():
                idx_copy(i + 1, nb).wait()

                @pl.when(i >= 1)
                def _():
                    store_copy(i - 1, nb).wait()
                gather_copy(nb).start()

                @pl.when(i + 2 < n_chunk)
                def _():
                    idx_copy(i + 2, b).start()

        def body(j, _):
            pipe_step(2 * j, 0, 1)
            pipe_step(2 * j + 1, 1, 0)
            return 0

        lax.fori_loop(0, n_chunk // 2, body, 0)

        store_copy(n_chunk - 2, (n_chunk - 2) % 2).wait()
        store_copy(n_chunk - 1, (n_chunk - 1) % 2).wait()

    return gather_kernel


def kernel(chars, table):
    b, s = chars.shape
    idx = chars.reshape(-1).astype(jnp.int32)
    out = _make_gather(b * s)(idx, table)
    return out.reshape(b, s, D)


# speedup vs baseline: 1.3816x; 1.0017x over previous
import functools

import jax
import jax.numpy as jnp
from jax import lax
from jax.experimental import pallas as pl
from jax.experimental.pallas import tpu as pltpu
from jax.experimental.pallas import tpu_sc as plsc

D = 32
NC, NS = 2, 16
NW = NC * NS
RPC = 4


def _make_gather(n_rows: int, seq: int, vocab: int):
    rows_per_w = n_rows // NW
    n_chunk = rows_per_w // RPC
    assert n_chunk % 2 == 0
    mesh = plsc.VectorSubcoreMesh(core_axis_name="c", subcore_axis_name="s")

    @functools.partial(
        pl.kernel,
        mesh=mesh,
        compiler_params=pltpu.CompilerParams(use_tc_tiling_on_sc=False),
        out_type=jax.ShapeDtypeStruct((n_rows, seq, D), jnp.float32),
        scratch_types=[
            pltpu.VMEM_SHARED((vocab, D), jnp.float32),
            pltpu.VMEM((RPC, seq), jnp.int32),
            pltpu.VMEM((RPC, seq), jnp.int32),
            pltpu.VMEM((RPC, seq, D), jnp.float32),
            pltpu.VMEM((RPC, seq, D), jnp.float32),
            pltpu.SemaphoreType.DMA,
            pltpu.SemaphoreType.DMA,
            pltpu.SemaphoreType.DMA,
            pltpu.SemaphoreType.DMA,
            pltpu.SemaphoreType.DMA,
            pltpu.SemaphoreType.DMA,
        ],
    )
    def gather_kernel(chars_hbm, table_hbm, out_hbm,
                      table_sh, idx0, idx1, rows0, rows1,
                      si0, si1, sg0, sg1, ss0, ss1):
        wid = lax.axis_index("s") * NC + lax.axis_index("c")
        wbase = wid * rows_per_w
        idx_v = (idx0, idx1)
        rows_v = (rows0, rows1)
        sem_i = (si0, si1)
        sem_g = (sg0, sg1)
        sem_s = (ss0, ss1)

        @pl.when(lax.axis_index("s") == 0)
        def _():
            pltpu.sync_copy(table_hbm, table_sh)
        plsc.subcore_barrier()

        def idx_copy(i, b):
            return pltpu.make_async_copy(
                chars_hbm.at[pl.ds(wbase + i * RPC, RPC)], idx_v[b], sem_i[b])

        def gather_copy(b, j):
            return pltpu.make_async_copy(
                table_sh.at[idx_v[b].at[j]], rows_v[b].at[j], sem_g[b])

        def gather_start(b):
            for j in range(RPC):
                gather_copy(b, j).start()

        def gather_wait(b):
            for j in range(RPC):
                gather_copy(b, j).wait()

        def store_copy(i, b):
            return pltpu.make_async_copy(
                rows_v[b], out_hbm.at[pl.ds(wbase + i * RPC, RPC)], sem_s[b])

        idx_copy(0, 0).start()
        idx_copy(1, 1).start()
        idx_copy(0, 0).wait()
        gather_start(0)

        def pipe_step(i, b, nb):
            gather_wait(b)
            store_copy(i, b).start()

            @pl.when(i + 1 < n_chunk)
            def _():
                idx_copy(i + 1, nb).wait()

                @pl.when(i >= 1)
                def _():
                    store_copy(i - 1, nb).wait()
                gather_start(nb)

                @pl.when(i + 2 < n_chunk)
                def _():
                    idx_copy(i + 2, b).start()

        def body(j, _):
            pipe_step(2 * j, 0, 1)
            pipe_step(2 * j + 1, 1, 0)
            return 0

        lax.fori_loop(0, n_chunk // 2, body, 0)

        store_copy(n_chunk - 2, (n_chunk - 2) % 2).wait()
        store_copy(n_chunk - 1, (n_chunk - 1) % 2).wait()

    return gather_kernel


def kernel(chars, table):
    b, s = chars.shape
    v, d = table.shape
    return _make_gather(b, s, v)(chars.astype(jnp.int32), table)


# speedup vs baseline: 1.3832x; 1.0012x over previous
import functools

import jax
import jax.numpy as jnp
from jax import lax
from jax.experimental import pallas as pl
from jax.experimental.pallas import tpu as pltpu
from jax.experimental.pallas import tpu_sc as plsc

D = 32
NC, NS = 2, 16
NW = NC * NS
RPC = 8


def _make_gather(n_rows: int, seq: int, vocab: int):
    rows_per_w = n_rows // NW
    n_chunk = rows_per_w // RPC
    assert n_chunk % 2 == 0
    mesh = plsc.VectorSubcoreMesh(core_axis_name="c", subcore_axis_name="s")

    @functools.partial(
        pl.kernel,
        mesh=mesh,
        compiler_params=pltpu.CompilerParams(use_tc_tiling_on_sc=False),
        out_type=jax.ShapeDtypeStruct((n_rows, seq, D), jnp.float32),
        scratch_types=[
            pltpu.VMEM_SHARED((vocab, D), jnp.float32),
            pltpu.VMEM((RPC, seq), jnp.int32),
            pltpu.VMEM((RPC, seq), jnp.int32),
            pltpu.VMEM((RPC, seq, D), jnp.float32),
            pltpu.VMEM((RPC, seq, D), jnp.float32),
            pltpu.SemaphoreType.DMA,
            pltpu.SemaphoreType.DMA,
            pltpu.SemaphoreType.DMA,
            pltpu.SemaphoreType.DMA,
            pltpu.SemaphoreType.DMA,
            pltpu.SemaphoreType.DMA,
        ],
    )
    def gather_kernel(chars_hbm, table_hbm, out_hbm,
                      table_sh, idx0, idx1, rows0, rows1,
                      si0, si1, sg0, sg1, ss0, ss1):
        wid = lax.axis_index("s") * NC + lax.axis_index("c")
        wbase = wid * rows_per_w
        idx_v = (idx0, idx1)
        rows_v = (rows0, rows1)
        sem_i = (si0, si1)
        sem_g = (sg0, sg1)
        sem_s = (ss0, ss1)

        @pl.when(lax.axis_index("s") == 0)
        def _():
            pltpu.sync_copy(table_hbm, table_sh)
        plsc.subcore_barrier()

        def idx_copy(i, b):
            return pltpu.make_async_copy(
                chars_hbm.at[pl.ds(wbase + i * RPC, RPC)], idx_v[b], sem_i[b])

        def gather_copy(b, j):
            return pltpu.make_async_copy(
                table_sh.at[idx_v[b].at[j]], rows_v[b].at[j], sem_g[b])

        def gather_start(b):
            for j in range(RPC):
                gather_copy(b, j).start()

        def gather_wait(b):
            for j in range(RPC):
                gather_copy(b, j).wait()

        def store_copy(i, b):
            return pltpu.make_async_copy(
                rows_v[b], out_hbm.at[pl.ds(wbase + i * RPC, RPC)], sem_s[b])

        idx_copy(0, 0).start()
        idx_copy(1, 1).start()
        idx_copy(0, 0).wait()
        gather_start(0)

        def pipe_step(i, b, nb):
            gather_wait(b)
            store_copy(i, b).start()

            @pl.when(i + 1 < n_chunk)
            def _():
                idx_copy(i + 1, nb).wait()

                @pl.when(i >= 1)
                def _():
                    store_copy(i - 1, nb).wait()
                gather_start(nb)

                @pl.when(i + 2 < n_chunk)
                def _():
                    idx_copy(i + 2, b).start()

        def body(j, _):
            pipe_step(2 * j, 0, 1)
            pipe_step(2 * j + 1, 1, 0)
            return 0

        lax.fori_loop(0, n_chunk // 2, body, 0)

        store_copy(n_chunk - 2, (n_chunk - 2) % 2).wait()
        store_copy(n_chunk - 1, (n_chunk - 1) % 2).wait()

    return gather_kernel


def kernel(chars, table):
    b, s = chars.shape
    v, d = table.shape
    return _make_gather(b, s, v)(chars.astype(jnp.int32), table)
